# baseline (speedup 1.0000x reference)
import jax
import jax.numpy as jnp
from jax.experimental import pallas as pl

N_CODEBOOKS = 14
VOCAB_P1 = 1025
LATENT = 8
D_MODEL = 1280
FAN_IN = N_CODEBOOKS * LATENT

BLK_M = 512


def _matmul_body(e_ref, wt_ref, b_ref, o_ref):
    o_ref[...] = (
        jnp.dot(e_ref[...], wt_ref[...], preferred_element_type=jnp.float32)
        + b_ref[...]
    )


def _project(e2d, Wt, b2d):
    m = e2d.shape[0]
    grid = (m // BLK_M,)
    return pl.pallas_call(
        _matmul_body,
        grid=grid,
        in_specs=[
            pl.BlockSpec((BLK_M, FAN_IN), lambda i: (i, 0)),
            pl.BlockSpec((FAN_IN, D_MODEL), lambda i: (0, 0)),
            pl.BlockSpec((1, D_MODEL), lambda i: (0, 0)),
        ],
        out_specs=pl.BlockSpec((BLK_M, D_MODEL), lambda i: (i, 0)),
        out_shape=jax.ShapeDtypeStruct((m, D_MODEL), jnp.float32),
    )(e2d, Wt, b2d)


def kernel(codes, tables, W, b):
    B, nc, S = codes.shape
    flat_tables = tables.reshape(nc * VOCAB_P1, LATENT)
    offs = (jnp.arange(nc, dtype=jnp.int32) * VOCAB_P1)[None, :, None]
    idx = (codes + offs).transpose(0, 2, 1).reshape(-1)
    e = jnp.take(flat_tables, idx, axis=0).reshape(B * S, FAN_IN)
    out = _project(e, W.T, b[None, :])
    return out.reshape(B, S, D_MODEL)
